# Initial kernel scaffold; baseline (speedup 1.0000x reference)
import jax
import jax.numpy as jnp
from jax.experimental import pallas as pl


def kernel(loc, conf):
    raise NotImplementedError("write your pallas kernel here")


# speedup vs baseline: 19.6506x; 19.6506x over previous
import math

import numpy as np
import jax
import jax.numpy as jnp
from jax.experimental import pallas as pl
from jax.experimental.pallas import tpu as pltpu

_INPUT_SIZE = [512, 1408]
_FM_SIZES = [[64, 176], [32, 88], [16, 44], [8, 22], [4, 11], [2, 6], [1, 3]]
_ANCHOR_SIZES = [0.04, 0.1, 0.26, 0.42, 0.58, 0.74, 0.9, 1.06]
_ASPECT_RATIOS = [[2], [2, 3], [2, 3], [2, 3], [2], [2], [2]]
_VAR0, _VAR1 = 0.1, 0.2
_PRE_NMS_TOPK = 1000
_MAX_DET = 200
_IOU_THRESH = 0.5
_SCORE_THRESH = 0.05
_NUM_CLASSES = 21


def _gen_anchors():
    input_aspect_ratio = float(_INPUT_SIZE[1]) / float(_INPUT_SIZE[0])
    steps = [(1.0 / h, 1.0 / w) for h, w in _FM_SIZES]
    anchors = []
    for i, (fh, fw) in enumerate(_FM_SIZES):
        min_size, max_size = _ANCHOR_SIZES[i], _ANCHOR_SIZES[i + 1]
        w, h = np.meshgrid(np.arange(fw), np.arange(fh))
        cx = (w + 0.5) * steps[i][1]
        cy = (h + 0.5) * steps[i][0]
        box = []
        for ar in _ASPECT_RATIOS[i]:
            box_short = min_size / math.sqrt(ar)
            box_long = min_size * math.sqrt(ar)
            box.append([box_long, box_short])
        box.append([math.sqrt(min_size * max_size)] * 2)
        anchor_num = len(_ASPECT_RATIOS[i]) + 1
        wh = np.tile(box, (fh, fw, 1, 1))
        cxcy = np.tile((cx, cy), (anchor_num, 1, 1, 1)).transpose(2, 3, 0, 1)
        b = np.concatenate((cxcy, wh), axis=3).reshape(-1, 4)
        b[:, 2] /= input_aspect_ratio
        anchors.append(b)
    return np.concatenate(anchors, axis=0).astype(np.float32)


_DB_NP = _gen_anchors()
_N = int(_DB_NP.shape[0])
_BLK = 1024
_NPAD = ((_N + _BLK - 1) // _BLK) * _BLK
_DB_PAD = np.zeros((_NPAD, 4), np.float32)
_DB_PAD[:_N] = _DB_NP
_K = _PRE_NMS_TOPK
_KPAD = 1024


def _pre_kernel(conf_ref, loc_ref, db_ref, score_ref, cls_ref, box_ref):
    c = conf_ref[...]
    m = jnp.max(c, axis=1, keepdims=True)
    iota = jax.lax.broadcasted_iota(jnp.int32, c.shape, 1)
    cls = jnp.min(jnp.where(c == m, iota, jnp.int32(2**30)), axis=1, keepdims=True)
    l = loc_ref[...]
    d = db_ref[...]
    wh = jnp.exp(l[:, 2:4] * _VAR1) * d[:, 2:4]
    cxcy = l[:, 0:2] * _VAR0 * d[:, 2:4] + d[:, 0:2]
    b = jnp.concatenate([cxcy - wh / 2.0, cxcy + wh / 2.0], axis=1)
    b = jnp.clip(b, 0.0, 1.0)
    score_ref[...] = m
    cls_ref[...] = cls
    box_ref[...] = b


def _nms_kernel(bx_ref, bt_ref, s_ref, keep_ref, mask_ref):
    x0c = bx_ref[:, 0:1]
    y0c = bx_ref[:, 1:2]
    x1c = bx_ref[:, 2:3]
    y1c = bx_ref[:, 3:4]
    x0r = bt_ref[0:1, :]
    y0r = bt_ref[1:2, :]
    x1r = bt_ref[2:3, :]
    y1r = bt_ref[3:4, :]
    w = jnp.maximum(jnp.minimum(x1c, x1r) - jnp.maximum(x0c, x0r), 0.0)
    h = jnp.maximum(jnp.minimum(y1c, y1r) - jnp.maximum(y0c, y0r), 0.0)
    inter = w * h
    areac = (x1c - x0c) * (y1c - y0c)
    arear = (x1r - x0r) * (y1r - y0r)
    iou = inter / (areac + arear - inter + 1e-12)
    ii = jax.lax.broadcasted_iota(jnp.int32, iou.shape, 0)
    jj = jax.lax.broadcasted_iota(jnp.int32, iou.shape, 1)
    mask_ref[...] = jnp.where((iou > _IOU_THRESH) & (jj > ii), 1.0, 0.0)
    keep0 = (s_ref[...] > _SCORE_THRESH).astype(jnp.float32)
    lane = jax.lax.broadcasted_iota(jnp.int32, (1, _KPAD), 1)

    def body(i, keep):
        row = mask_ref[pl.ds(i, 1), :]
        ki = jnp.sum(jnp.where(lane == i, keep, 0.0))
        return keep * (1.0 - ki * row)

    keep = jax.lax.fori_loop(0, _K, body, keep0)
    keep_ref[...] = keep


def _pre_call(conf_s, loc_pad, db):
    grid = _NPAD // _BLK
    return pl.pallas_call(
        _pre_kernel,
        grid=(grid,),
        in_specs=[
            pl.BlockSpec((_BLK, _NUM_CLASSES), lambda i: (i, 0)),
            pl.BlockSpec((_BLK, 4), lambda i: (i, 0)),
            pl.BlockSpec((_BLK, 4), lambda i: (i, 0)),
        ],
        out_specs=[
            pl.BlockSpec((_BLK, 1), lambda i: (i, 0)),
            pl.BlockSpec((_BLK, 1), lambda i: (i, 0)),
            pl.BlockSpec((_BLK, 4), lambda i: (i, 0)),
        ],
        out_shape=[
            jax.ShapeDtypeStruct((_NPAD, 1), jnp.float32),
            jax.ShapeDtypeStruct((_NPAD, 1), jnp.int32),
            jax.ShapeDtypeStruct((_NPAD, 4), jnp.float32),
        ],
    )(conf_s, loc_pad, db)


def _nms_call(b_pad, bt, s_pad):
    return pl.pallas_call(
        _nms_kernel,
        out_shape=jax.ShapeDtypeStruct((1, _KPAD), jnp.float32),
        scratch_shapes=[pltpu.VMEM((_KPAD, _KPAD), jnp.float32)],
    )(b_pad, bt, s_pad)


@jax.jit
def kernel(loc, conf):
    conf_s = jax.nn.sigmoid(conf)
    conf_pad = jnp.pad(conf_s, ((0, _NPAD - _N), (0, 0)))
    loc_pad = jnp.pad(loc, ((0, _NPAD - _N), (0, 0)))
    db = jnp.asarray(_DB_PAD)
    score2d, cls2d, boxes_full = _pre_call(conf_pad, loc_pad, db)
    scores_all = score2d[:_N, 0]
    classes_all = cls2d[:_N, 0]

    s_top, idx = jax.lax.top_k(scores_all, _K)
    b_top = jnp.take(boxes_full[:_N], idx, axis=0)
    c_top = jnp.take(classes_all, idx, axis=0)

    b_pad = jnp.pad(b_top, ((0, _KPAD - _K), (0, 0)))
    bt = b_pad.T
    s_pad = jnp.pad(s_top, (0, _KPAD - _K), constant_values=-1.0)[None, :]

    keep2d = _nms_call(b_pad, bt, s_pad)
    keep = keep2d[0, :_K]

    masked = jnp.where(keep > 0.0, s_top, -1.0)
    out_s, sel = jax.lax.top_k(masked, _MAX_DET)
    out_b = jnp.take(b_top, sel, axis=0)
    out_c = jnp.take(c_top, sel, axis=0)
    keep_idx = jnp.take(idx, sel, axis=0)
    num = jnp.minimum(jnp.sum(keep.astype(jnp.int32)), _MAX_DET)
    valid = jnp.arange(_MAX_DET) < num
    out_s = jnp.where(valid, out_s, 0.0)
    out_b = out_b * valid[:, None].astype(out_b.dtype)
    return out_b, out_s, out_c, keep_idx, num


# speedup vs baseline: 33.9194x; 1.7261x over previous
import math

import numpy as np
import jax
import jax.numpy as jnp
from jax.experimental import pallas as pl
from jax.experimental.pallas import tpu as pltpu

_INPUT_SIZE = [512, 1408]
_FM_SIZES = [[64, 176], [32, 88], [16, 44], [8, 22], [4, 11], [2, 6], [1, 3]]
_ANCHOR_SIZES = [0.04, 0.1, 0.26, 0.42, 0.58, 0.74, 0.9, 1.06]
_ASPECT_RATIOS = [[2], [2, 3], [2, 3], [2, 3], [2], [2], [2]]
_VAR0, _VAR1 = 0.1, 0.2
_PRE_NMS_TOPK = 1000
_MAX_DET = 200
_IOU_THRESH = 0.5
_SCORE_THRESH = 0.05
_NUM_CLASSES = 21


def _gen_anchors():
    input_aspect_ratio = float(_INPUT_SIZE[1]) / float(_INPUT_SIZE[0])
    steps = [(1.0 / h, 1.0 / w) for h, w in _FM_SIZES]
    anchors = []
    for i, (fh, fw) in enumerate(_FM_SIZES):
        min_size, max_size = _ANCHOR_SIZES[i], _ANCHOR_SIZES[i + 1]
        w, h = np.meshgrid(np.arange(fw), np.arange(fh))
        cx = (w + 0.5) * steps[i][1]
        cy = (h + 0.5) * steps[i][0]
        box = []
        for ar in _ASPECT_RATIOS[i]:
            box_short = min_size / math.sqrt(ar)
            box_long = min_size * math.sqrt(ar)
            box.append([box_long, box_short])
        box.append([math.sqrt(min_size * max_size)] * 2)
        anchor_num = len(_ASPECT_RATIOS[i]) + 1
        wh = np.tile(box, (fh, fw, 1, 1))
        cxcy = np.tile((cx, cy), (anchor_num, 1, 1, 1)).transpose(2, 3, 0, 1)
        b = np.concatenate((cxcy, wh), axis=3).reshape(-1, 4)
        b[:, 2] /= input_aspect_ratio
        anchors.append(b)
    return np.concatenate(anchors, axis=0).astype(np.float32)


_DB_NP = _gen_anchors()
_N = int(_DB_NP.shape[0])
_BLK = 1024
_NPAD = ((_N + _BLK - 1) // _BLK) * _BLK
_DB_PAD = np.zeros((_NPAD, 4), np.float32)
_DB_PAD[:_N] = _DB_NP
_K = _PRE_NMS_TOPK
_KPAD = 1024


def _pre_kernel(conf_ref, loc_ref, db_ref, score_ref, cls_ref, box_ref):
    c = conf_ref[...]
    m = jnp.max(c, axis=1, keepdims=True)
    iota = jax.lax.broadcasted_iota(jnp.int32, c.shape, 1)
    cls = jnp.min(jnp.where(c == m, iota, jnp.int32(2**30)), axis=1, keepdims=True)
    l = loc_ref[...]
    d = db_ref[...]
    wh = jnp.exp(l[:, 2:4] * _VAR1) * d[:, 2:4]
    cxcy = l[:, 0:2] * _VAR0 * d[:, 2:4] + d[:, 0:2]
    b = jnp.concatenate([cxcy - wh / 2.0, cxcy + wh / 2.0], axis=1)
    b = jnp.clip(b, 0.0, 1.0)
    score_ref[...] = m
    cls_ref[...] = cls
    box_ref[...] = b


def _nms_kernel(bx_ref, bt_ref, s_ref, keep_ref, mask_ref):
    x0c = bx_ref[:, 0:1]
    y0c = bx_ref[:, 1:2]
    x1c = bx_ref[:, 2:3]
    y1c = bx_ref[:, 3:4]
    x0r = bt_ref[0:1, :]
    y0r = bt_ref[1:2, :]
    x1r = bt_ref[2:3, :]
    y1r = bt_ref[3:4, :]
    w = jnp.maximum(jnp.minimum(x1c, x1r) - jnp.maximum(x0c, x0r), 0.0)
    h = jnp.maximum(jnp.minimum(y1c, y1r) - jnp.maximum(y0c, y0r), 0.0)
    inter = w * h
    areac = (x1c - x0c) * (y1c - y0c)
    arear = (x1r - x0r) * (y1r - y0r)
    iou = inter / (areac + arear - inter + 1e-12)
    ii = jax.lax.broadcasted_iota(jnp.int32, iou.shape, 0)
    jj = jax.lax.broadcasted_iota(jnp.int32, iou.shape, 1)
    mask_ref[...] = jnp.where((iou > _IOU_THRESH) & (jj > ii), 1.0, 0.0).astype(
        jnp.bfloat16
    )
    init = (s_ref[...] > _SCORE_THRESH).astype(jnp.float32)

    def cond(c):
        return c[1]

    def body(c):
        keep, _ = c
        sup = jax.lax.dot_general(
            keep.astype(jnp.bfloat16),
            mask_ref[...],
            (((1,), (0,)), ((), ())),
            preferred_element_type=jnp.float32,
        )
        new = init * (sup == 0.0).astype(jnp.float32)
        return new, jnp.any(new != keep)

    keep, _ = jax.lax.while_loop(cond, body, (init, True))
    keep_ref[...] = keep


def _pre_call(conf_s, loc_pad, db):
    grid = _NPAD // _BLK
    return pl.pallas_call(
        _pre_kernel,
        grid=(grid,),
        in_specs=[
            pl.BlockSpec((_BLK, _NUM_CLASSES), lambda i: (i, 0)),
            pl.BlockSpec((_BLK, 4), lambda i: (i, 0)),
            pl.BlockSpec((_BLK, 4), lambda i: (i, 0)),
        ],
        out_specs=[
            pl.BlockSpec((_BLK, 1), lambda i: (i, 0)),
            pl.BlockSpec((_BLK, 1), lambda i: (i, 0)),
            pl.BlockSpec((_BLK, 4), lambda i: (i, 0)),
        ],
        out_shape=[
            jax.ShapeDtypeStruct((_NPAD, 1), jnp.float32),
            jax.ShapeDtypeStruct((_NPAD, 1), jnp.int32),
            jax.ShapeDtypeStruct((_NPAD, 4), jnp.float32),
        ],
    )(conf_s, loc_pad, db)


def _nms_call(b_pad, bt, s_pad):
    return pl.pallas_call(
        _nms_kernel,
        out_shape=jax.ShapeDtypeStruct((1, _KPAD), jnp.float32),
        scratch_shapes=[pltpu.VMEM((_KPAD, _KPAD), jnp.bfloat16)],
    )(b_pad, bt, s_pad)


@jax.jit
def kernel(loc, conf):
    conf_s = jax.nn.sigmoid(conf)
    db = jnp.asarray(_DB_PAD)
    score2d, cls2d, boxes_full = _pre_call(conf_s, loc, db)
    scores_all = score2d[:_N, 0]
    classes_all = cls2d[:_N, 0]

    s_top, idx = jax.lax.top_k(scores_all, _K)
    b_top = jnp.take(boxes_full[:_N], idx, axis=0)
    c_top = jnp.take(classes_all, idx, axis=0)

    b_pad = jnp.pad(b_top, ((0, _KPAD - _K), (0, 0)))
    bt = b_pad.T
    s_pad = jnp.pad(s_top, (0, _KPAD - _K), constant_values=-1.0)[None, :]

    keep2d = _nms_call(b_pad, bt, s_pad)
    keep = keep2d[0, :_K]

    masked = jnp.where(keep > 0.0, s_top, -1.0)
    out_s, sel = jax.lax.top_k(masked, _MAX_DET)
    out_b = jnp.take(b_top, sel, axis=0)
    out_c = jnp.take(c_top, sel, axis=0)
    keep_idx = jnp.take(idx, sel, axis=0)
    num = jnp.minimum(jnp.sum(keep.astype(jnp.int32)), _MAX_DET)
    valid = jnp.arange(_MAX_DET) < num
    out_s = jnp.where(valid, out_s, 0.0)
    out_b = out_b * valid[:, None].astype(out_b.dtype)
    return out_b, out_s, out_c, keep_idx, num
